# baseline (speedup 1.0000x reference)
import functools

import jax
import jax.numpy as jnp
from jax import lax
from jax.experimental import pallas as pl
from jax.experimental.pallas import tpu as pltpu
from jax.experimental.pallas import tpu_sc as plsc

_B, _C, _H, _W = 2, 96, 384, 384
_HW = _H * _W
_NT = 16
_NC = 2
_CB = 8
_NCB = _C // _CB
_PT = _HW // _NT
_RC = 2
_CHUNK = _RC * _W
_NCHUNK = _PT // _CHUNK
_G128 = _CHUNK // 128
_G16 = _W // 16


_HB = 96


def _taps_body(flow_ref, idx_ref, wgt_ref):
    h = pl.program_id(1)
    fx = flow_ref[0, 0]
    fy = flow_ref[0, 1]
    shp = fx.shape
    xi0 = lax.broadcasted_iota(jnp.int32, shp, 1)
    yi0 = lax.broadcasted_iota(jnp.int32, shp, 0) + h * _HB
    x = xi0.astype(jnp.float32)
    y = yi0.astype(jnp.float32)
    own = yi0 * _W + xi0
    out_x = x + fx
    out_y = y + fy
    x0f = jnp.floor(out_x)
    y0f = jnp.floor(out_y)
    x0 = x0f.astype(jnp.int32)
    y0 = y0f.astype(jnp.int32)
    ax = out_x - x0f
    ay = out_y - y0f
    bx = 1.0 - ax
    by = 1.0 - ay
    taps = (
        (x0, y0, bx * by),
        (x0 + 1, y0, ax * by),
        (x0, y0 + 1, bx * ay),
        (x0 + 1, y0 + 1, ax * ay),
    )
    for t, (xi, yi, w) in enumerate(taps):
        valid = (xi >= 0) & (xi < _W) & (yi >= 0) & (yi < _H)
        idx_ref[0, t] = jnp.where(valid, yi * _W + xi, own)
        wgt_ref[0, t] = jnp.where(valid, w, 0.0)


def _taps(flow):
    return pl.pallas_call(
        _taps_body,
        grid=(_B, _H // _HB),
        in_specs=[pl.BlockSpec((1, 2, _HB, _W), lambda b, h: (b, 0, h, 0))],
        out_specs=[
            pl.BlockSpec((1, 4, _HB, _W), lambda b, h: (b, 0, h, 0)),
            pl.BlockSpec((1, 4, _HB, _W), lambda b, h: (b, 0, h, 0)),
        ],
        out_shape=[
            jax.ShapeDtypeStruct((_B, 4, _H, _W), jnp.int32),
            jax.ShapeDtypeStruct((_B, 4, _H, _W), jnp.float32),
        ],
    )(flow)


def _splat_sc_body(frame, idxr, wgt, zeros, out, table, vraw_a, vraw_b, wb_a,
                   wb_b, idxb, vals_0, vals_1, sem_in_a, sem_in_b, sem_sc_0,
                   sem_sc_1, sem_misc):
    cid = lax.axis_index("c")
    sid = lax.axis_index("s")
    tile_p0 = sid * _PT
    tile_r0 = sid * (_PT // _W)

    i16 = lax.iota(jnp.int32, 16)
    cols = [jnp.full((16,), ch, jnp.int32) for ch in range(_CB)]
    vraws = (vraw_a, vraw_b)
    wbs = (wb_a, wb_b)
    valss = (vals_0, vals_1)
    sems_in = (sem_in_a, sem_in_b)
    sems_sc = (sem_sc_0, sem_sc_1)

    def task_body(i, carry):
        cb = 2 * i + cid

        @pl.when(i == 0)
        def _():
            for k in range(_NCHUNK):
                pltpu.async_copy(
                    zeros, table.at[pl.ds(tile_p0 + k * _CHUNK, _CHUNK), :],
                    sem_misc)
            for k in range(_NCHUNK):
                pltpu.make_async_copy(
                    zeros, table.at[pl.ds(tile_p0, _CHUNK), :],
                    sem_misc).wait()

        plsc.subcore_barrier()

        def issue_inputs(k, kk):
            pbase = tile_p0 + k * _CHUNK
            row0 = tile_r0 + k * _RC
            pltpu.async_copy(
                frame.at[pl.ds(cb * _CB, _CB), pl.ds(row0, _RC), :],
                vraws[kk], sems_in[kk])
            pltpu.async_copy(
                wgt.at[:, pl.ds(pbase, _CHUNK)], wbs[kk], sems_in[kk])
            pltpu.async_copy(
                idxr.at[:, pl.ds(pbase, _CHUNK)], idxb.at[k % 3],
                sems_in[kk])

        def wait_inputs(kk):
            pltpu.make_async_copy(
                frame.at[pl.ds(cb * _CB, _CB), pl.ds(tile_r0, _RC), :],
                vraws[kk], sems_in[kk]).wait()
            pltpu.make_async_copy(
                wgt.at[:, pl.ds(tile_p0, _CHUNK)], wbs[kk],
                sems_in[kk]).wait()
            pltpu.make_async_copy(
                idxr.at[:, pl.ds(tile_p0, _CHUNK)], idxb.at[0],
                sems_in[kk]).wait()

        def drain_tap(t):
            pltpu.make_async_copy(
                valss[t % 2], table.at[idxb.at[0, t]],
                sems_sc[t % 2]).wait()

        def compute_tap(kk, t):
            vraw, wb, vals = vraws[kk], wbs[kk], valss[t % 2]

            for r in range(_RC):
                def cgroup(g, cc, r=r):
                    col = g * 16
                    rb = r * _W + col
                    rowv = i16 + rb
                    wv = wb[t, pl.ds(rb, 16)]
                    for ch in range(_CB):
                        v = vraw[ch, r, pl.ds(col, 16)]
                        plsc.store_scatter(vals, [rowv, cols[ch]], v * wv)
                    return cc

                lax.fori_loop(0, _G16, cgroup, 0)

        def fire_tap(k, t):
            pltpu.async_copy(
                valss[t % 2],
                table.at[idxb.at[k % 3, t]],
                sems_sc[t % 2],
                add=True)

        issue_inputs(0, 0)
        issue_inputs(1, 1)

        def chunk_pair(m, c):
            for kk in range(2):
                k = 2 * m + kk
                wait_inputs(kk)
                for t in range(4):
                    if t < 2:
                        @pl.when(k >= 1)
                        def _():
                            drain_tap(t)
                    else:
                        drain_tap(t)
                    compute_tap(kk, t)
                    fire_tap(k, t)

                @pl.when(k + 2 < _NCHUNK)
                def _():
                    issue_inputs(k + 2, kk)
            return c

        lax.fori_loop(0, _NCHUNK // 2, chunk_pair, 0)
        drain_tap(2)
        drain_tap(3)
        plsc.subcore_barrier()

        def issue_tin(k, kk):
            pltpu.async_copy(
                table.at[pl.ds(tile_p0 + k * _CHUNK, _CHUNK), :],
                valss[kk], sems_in[kk])

        def wait_tin(kk):
            pltpu.make_async_copy(
                table.at[pl.ds(tile_p0, _CHUNK), :], valss[kk],
                sems_in[kk]).wait()

        def fire_tout(k, kk):
            pltpu.async_copy(
                vraws[kk],
                out.at[pl.ds(cb * _CB, _CB),
                       pl.ds(tile_r0 + k * _RC, _RC), :],
                sems_sc[kk])

        def wait_tout(kk):
            pltpu.make_async_copy(
                vraws[kk],
                out.at[pl.ds(cb * _CB, _CB), pl.ds(tile_r0, _RC), :],
                sems_sc[kk]).wait()

        issue_tin(0, 0)

        def drain_pair(m, c):
            for kk in range(2):
                k = 2 * m + kk

                @pl.when(k + 1 < _NCHUNK)
                def _():
                    issue_tin(k + 1, 1 - kk)

                wait_tin(kk)
                pltpu.async_copy(
                    zeros,
                    table.at[pl.ds(tile_p0 + k * _CHUNK, _CHUNK), :],
                    sem_misc)

                @pl.when(k >= 2)
                def _():
                    wait_tout(kk)

                obuf, obuft = valss[kk], vraws[kk]
                for r in range(_RC):
                    def tgroup(g, cc, r=r):
                        col = g * 16
                        rows = i16 + (r * _W + col)
                        for ch in range(_CB):
                            v = plsc.load_gather(obuf, [rows, cols[ch]])
                            obuft[ch, r, pl.ds(col, 16)] = v
                        return cc

                    lax.fori_loop(0, _G16, tgroup, 0)
                fire_tout(k, kk)
            return c

        lax.fori_loop(0, _NCHUNK // 2, drain_pair, 0)
        wait_tout(0)
        wait_tout(1)
        for k in range(_NCHUNK):
            pltpu.make_async_copy(
                zeros, table.at[pl.ds(tile_p0, _CHUNK), :], sem_misc).wait()
        plsc.subcore_barrier()
        return carry

    lax.fori_loop(0, _NCB // _NC, task_body, 0)


_splat_sc = functools.partial(
    pl.kernel,
    out_type=jax.ShapeDtypeStruct((_C, _H, _W), jnp.float32),
    mesh=plsc.VectorSubcoreMesh(core_axis_name="c", subcore_axis_name="s"),
    scratch_types=[
        pltpu.VMEM_SHARED((_HW, _CB), jnp.float32),
        pltpu.VMEM((_CB, _RC, _W), jnp.float32),
        pltpu.VMEM((_CB, _RC, _W), jnp.float32),
        pltpu.VMEM((4, _CHUNK), jnp.float32),
        pltpu.VMEM((4, _CHUNK), jnp.float32),
        pltpu.VMEM((3, 4, _CHUNK), jnp.int32),
        pltpu.VMEM((_CHUNK, _CB), jnp.float32),
        pltpu.VMEM((_CHUNK, _CB), jnp.float32),
        pltpu.SemaphoreType.DMA,
        pltpu.SemaphoreType.DMA,
        pltpu.SemaphoreType.DMA,
        pltpu.SemaphoreType.DMA,
        pltpu.SemaphoreType.DMA,
    ],
    compiler_params=pltpu.CompilerParams(
        needs_layout_passes=False, use_tc_tiling_on_sc=False),
)(_splat_sc_body)


@jax.jit
def kernel(frame, flow):
    idx, wgt = _taps(flow)
    zeros = jnp.zeros((_CHUNK, _CB), jnp.float32)
    outs = [
        _splat_sc(frame[b], idx[b].reshape(4, _HW), wgt[b].reshape(4, _HW),
                  zeros) for b in range(_B)
    ]
    return jnp.stack(outs)


# speedup vs baseline: 1.0433x; 1.0433x over previous
import functools

import jax
import jax.numpy as jnp
from jax import lax
from jax.experimental import pallas as pl
from jax.experimental.pallas import tpu as pltpu
from jax.experimental.pallas import tpu_sc as plsc

_B, _C, _H, _W = 2, 96, 384, 384
_HW = _H * _W
_NT = 16
_NC = 2
_CB = 8
_NCB = _C // _CB
_PT = _HW // _NT
_RC = 2
_CHUNK = _RC * _W
_NCHUNK = _PT // _CHUNK
_G128 = _CHUNK // 128
_G16 = _W // 16


_HB = 96


def _taps_body(flow_ref, idx_ref, wgt_ref):
    h = pl.program_id(1)
    fx = flow_ref[0, 0]
    fy = flow_ref[0, 1]
    shp = fx.shape
    xi0 = lax.broadcasted_iota(jnp.int32, shp, 1)
    yi0 = lax.broadcasted_iota(jnp.int32, shp, 0) + h * _HB
    x = xi0.astype(jnp.float32)
    y = yi0.astype(jnp.float32)
    own = yi0 * _W + xi0
    out_x = x + fx
    out_y = y + fy
    x0f = jnp.floor(out_x)
    y0f = jnp.floor(out_y)
    x0 = x0f.astype(jnp.int32)
    y0 = y0f.astype(jnp.int32)
    ax = out_x - x0f
    ay = out_y - y0f
    bx = 1.0 - ax
    by = 1.0 - ay
    taps = (
        (x0, y0, bx * by),
        (x0 + 1, y0, ax * by),
        (x0, y0 + 1, bx * ay),
        (x0 + 1, y0 + 1, ax * ay),
    )
    for t, (xi, yi, w) in enumerate(taps):
        valid = (xi >= 0) & (xi < _W) & (yi >= 0) & (yi < _H)
        idx_ref[0, t] = jnp.where(valid, yi * _W + xi, own)
        wgt_ref[0, t] = jnp.where(valid, w, 0.0)


def _taps(flow):
    return pl.pallas_call(
        _taps_body,
        grid=(_B, _H // _HB),
        in_specs=[pl.BlockSpec((1, 2, _HB, _W), lambda b, h: (b, 0, h, 0))],
        out_specs=[
            pl.BlockSpec((1, 4, _HB, _W), lambda b, h: (b, 0, h, 0)),
            pl.BlockSpec((1, 4, _HB, _W), lambda b, h: (b, 0, h, 0)),
        ],
        out_shape=[
            jax.ShapeDtypeStruct((_B, 4, _H, _W), jnp.int32),
            jax.ShapeDtypeStruct((_B, 4, _H, _W), jnp.float32),
        ],
    )(flow)


def _splat_sc_body(frame, idxr, wgt, zeros, out, table, vraw_a, vraw_b, wb_a,
                   wb_b, idxb, vals_0, vals_1, sem_in_a, sem_in_b, sem_sc_0,
                   sem_sc_1, sem_misc):
    cid = lax.axis_index("c")
    sid = lax.axis_index("s")
    tile_p0 = sid * _PT
    tile_r0 = sid * (_PT // _W)

    i16 = lax.iota(jnp.int32, 16)
    cols = [jnp.full((16,), ch, jnp.int32) for ch in range(_CB)]
    vraws = (vraw_a, vraw_b)
    wbs = (wb_a, wb_b)
    valss = (vals_0, vals_1)
    sems_in = (sem_in_a, sem_in_b)
    sems_sc = (sem_sc_0, sem_sc_1)

    def task_body(i, carry):
        b = i // (_NCB // _NC)
        cb = 2 * lax.rem(i, _NCB // _NC) + cid

        @pl.when(i == 0)
        def _():
            for k in range(_NCHUNK):
                pltpu.async_copy(
                    zeros, table.at[pl.ds(tile_p0 + k * _CHUNK, _CHUNK), :],
                    sem_misc)
            for k in range(_NCHUNK):
                pltpu.make_async_copy(
                    zeros, table.at[pl.ds(tile_p0, _CHUNK), :],
                    sem_misc).wait()

        def issue_inputs(k, kk):
            pbase = tile_p0 + k * _CHUNK
            row0 = tile_r0 + k * _RC
            pltpu.async_copy(
                frame.at[b, pl.ds(cb * _CB, _CB), pl.ds(row0, _RC), :],
                vraws[kk], sems_in[kk])
            pltpu.async_copy(
                wgt.at[b, :, pl.ds(pbase, _CHUNK)], wbs[kk], sems_in[kk])
            pltpu.async_copy(
                idxr.at[b, :, pl.ds(pbase, _CHUNK)], idxb.at[k % 3],
                sems_in[kk])

        def wait_inputs(kk):
            pltpu.make_async_copy(
                frame.at[b, pl.ds(cb * _CB, _CB), pl.ds(tile_r0, _RC), :],
                vraws[kk], sems_in[kk]).wait()
            pltpu.make_async_copy(
                wgt.at[b, :, pl.ds(tile_p0, _CHUNK)], wbs[kk],
                sems_in[kk]).wait()
            pltpu.make_async_copy(
                idxr.at[b, :, pl.ds(tile_p0, _CHUNK)], idxb.at[0],
                sems_in[kk]).wait()

        def drain_tap(t):
            pltpu.make_async_copy(
                valss[t % 2], table.at[idxb.at[0, t]],
                sems_sc[t % 2]).wait()

        def compute_tap(kk, t):
            vraw, wb, vals = vraws[kk], wbs[kk], valss[t % 2]

            for r in range(_RC):
                def cgroup(g, cc, r=r):
                    col = g * 16
                    rb = r * _W + col
                    rowv = i16 + rb
                    wv = wb[t, pl.ds(rb, 16)]
                    for ch in range(_CB):
                        v = vraw[ch, r, pl.ds(col, 16)]
                        plsc.store_scatter(vals, [rowv, cols[ch]], v * wv)
                    return cc

                lax.fori_loop(0, _G16, cgroup, 0)

        def fire_tap(k, t):
            pltpu.async_copy(
                valss[t % 2],
                table.at[idxb.at[k % 3, t]],
                sems_sc[t % 2],
                add=True)

        issue_inputs(0, 0)
        issue_inputs(1, 1)
        plsc.subcore_barrier()

        def chunk_pair(m, c):
            for kk in range(2):
                k = 2 * m + kk
                wait_inputs(kk)
                for t in range(4):
                    if t < 2:
                        @pl.when(k >= 1)
                        def _():
                            drain_tap(t)
                    else:
                        drain_tap(t)
                    compute_tap(kk, t)
                    fire_tap(k, t)

                @pl.when(k + 2 < _NCHUNK)
                def _():
                    issue_inputs(k + 2, kk)
            return c

        lax.fori_loop(0, _NCHUNK // 2, chunk_pair, 0)
        drain_tap(2)
        drain_tap(3)
        plsc.subcore_barrier()

        def issue_tin(k, kk):
            pltpu.async_copy(
                table.at[pl.ds(tile_p0 + k * _CHUNK, _CHUNK), :],
                valss[kk], sems_in[kk])

        def wait_tin(kk):
            pltpu.make_async_copy(
                table.at[pl.ds(tile_p0, _CHUNK), :], valss[kk],
                sems_in[kk]).wait()

        def fire_tout(k, kk):
            pltpu.async_copy(
                vraws[kk],
                out.at[b, pl.ds(cb * _CB, _CB),
                       pl.ds(tile_r0 + k * _RC, _RC), :],
                sems_sc[kk])

        def wait_tout(kk):
            pltpu.make_async_copy(
                vraws[kk],
                out.at[b, pl.ds(cb * _CB, _CB), pl.ds(tile_r0, _RC), :],
                sems_sc[kk]).wait()

        issue_tin(0, 0)

        def drain_pair(m, c):
            for kk in range(2):
                k = 2 * m + kk

                @pl.when(k + 1 < _NCHUNK)
                def _():
                    issue_tin(k + 1, 1 - kk)

                wait_tin(kk)
                pltpu.async_copy(
                    zeros,
                    table.at[pl.ds(tile_p0 + k * _CHUNK, _CHUNK), :],
                    sem_misc)

                @pl.when(k >= 2)
                def _():
                    wait_tout(kk)

                obuf, obuft = valss[kk], vraws[kk]
                for r in range(_RC):
                    def tgroup(g, cc, r=r):
                        col = g * 16
                        rows = i16 + (r * _W + col)
                        for ch in range(_CB):
                            v = plsc.load_gather(obuf, [rows, cols[ch]])
                            obuft[ch, r, pl.ds(col, 16)] = v
                        return cc

                    lax.fori_loop(0, _G16, tgroup, 0)
                fire_tout(k, kk)
            return c

        lax.fori_loop(0, _NCHUNK // 2, drain_pair, 0)
        wait_tout(0)
        wait_tout(1)
        for k in range(_NCHUNK):
            pltpu.make_async_copy(
                zeros, table.at[pl.ds(tile_p0, _CHUNK), :], sem_misc).wait()
        return carry

    lax.fori_loop(0, _B * _NCB // _NC, task_body, 0)


_splat_sc = functools.partial(
    pl.kernel,
    out_type=jax.ShapeDtypeStruct((_B, _C, _H, _W), jnp.float32),
    mesh=plsc.VectorSubcoreMesh(core_axis_name="c", subcore_axis_name="s"),
    scratch_types=[
        pltpu.VMEM_SHARED((_HW, _CB), jnp.float32),
        pltpu.VMEM((_CB, _RC, _W), jnp.float32),
        pltpu.VMEM((_CB, _RC, _W), jnp.float32),
        pltpu.VMEM((4, _CHUNK), jnp.float32),
        pltpu.VMEM((4, _CHUNK), jnp.float32),
        pltpu.VMEM((3, 4, _CHUNK), jnp.int32),
        pltpu.VMEM((_CHUNK, _CB), jnp.float32),
        pltpu.VMEM((_CHUNK, _CB), jnp.float32),
        pltpu.SemaphoreType.DMA,
        pltpu.SemaphoreType.DMA,
        pltpu.SemaphoreType.DMA,
        pltpu.SemaphoreType.DMA,
        pltpu.SemaphoreType.DMA,
    ],
    compiler_params=pltpu.CompilerParams(
        needs_layout_passes=False, use_tc_tiling_on_sc=False),
)(_splat_sc_body)


@jax.jit
def kernel(frame, flow):
    idx, wgt = _taps(flow)
    return _splat_sc(
        frame,
        idx.reshape(_B, 4, _HW),
        wgt.reshape(_B, 4, _HW),
        jnp.zeros((_CHUNK, _CB), jnp.float32),
    )


# speedup vs baseline: 1.1899x; 1.1405x over previous
import functools

import jax
import jax.numpy as jnp
from jax import lax
from jax.experimental import pallas as pl
from jax.experimental.pallas import tpu as pltpu
from jax.experimental.pallas import tpu_sc as plsc

_B, _C, _H, _W = 2, 96, 384, 384
_HW = _H * _W
_NT = 16
_NC = 2
_CB = 8
_NCB = _C // _CB
_PT = _HW // _NT
_RC = 2
_CHUNK = _RC * _W
_NCHUNK = _PT // _CHUNK
_G128 = _CHUNK // 128
_G16 = _W // 16


_HB = 96


def _taps_body(flow_ref, idx_ref, wgt_ref):
    h = pl.program_id(1)
    fx = flow_ref[0, 0]
    fy = flow_ref[0, 1]
    shp = fx.shape
    xi0 = lax.broadcasted_iota(jnp.int32, shp, 1)
    yi0 = lax.broadcasted_iota(jnp.int32, shp, 0) + h * _HB
    x = xi0.astype(jnp.float32)
    y = yi0.astype(jnp.float32)
    own = yi0 * _W + xi0
    out_x = x + fx
    out_y = y + fy
    x0f = jnp.floor(out_x)
    y0f = jnp.floor(out_y)
    x0 = x0f.astype(jnp.int32)
    y0 = y0f.astype(jnp.int32)
    ax = out_x - x0f
    ay = out_y - y0f
    bx = 1.0 - ax
    by = 1.0 - ay
    taps = (
        (x0, y0, bx * by),
        (x0 + 1, y0, ax * by),
        (x0, y0 + 1, bx * ay),
        (x0 + 1, y0 + 1, ax * ay),
    )
    for t, (xi, yi, w) in enumerate(taps):
        valid = (xi >= 0) & (xi < _W) & (yi >= 0) & (yi < _H)
        idx_ref[0, t] = jnp.where(valid, yi * _W + xi, own)
        wgt_ref[0, t] = jnp.where(valid, w, 0.0)


def _taps(flow):
    return pl.pallas_call(
        _taps_body,
        grid=(_B, _H // _HB),
        in_specs=[pl.BlockSpec((1, 2, _HB, _W), lambda b, h: (b, 0, h, 0))],
        out_specs=[
            pl.BlockSpec((1, 4, _HB, _W), lambda b, h: (b, 0, h, 0)),
            pl.BlockSpec((1, 4, _HB, _W), lambda b, h: (b, 0, h, 0)),
        ],
        out_shape=[
            jax.ShapeDtypeStruct((_B, 4, _H, _W), jnp.int32),
            jax.ShapeDtypeStruct((_B, 4, _H, _W), jnp.float32),
        ],
    )(flow)


def _splat_sc_body(frame, idxr, wgt, zeros, out, table, vraw_a, vraw_b, wb_a,
                   wb_b, idxb, vals_0, vals_1, sem_in_a, sem_in_b, sem_sc_0,
                   sem_sc_1, sem_misc):
    cid = lax.axis_index("c")
    sid = lax.axis_index("s")
    tile_p0 = sid * _PT
    tile_r0 = sid * (_PT // _W)

    i16 = lax.iota(jnp.int32, 16)
    cols = [jnp.full((16,), ch, jnp.int32) for ch in range(_CB)]
    vraws = (vraw_a, vraw_b)
    wbs = (wb_a, wb_b)
    valss = (vals_0, vals_1)
    sems_in = (sem_in_a, sem_in_b)
    sems_sc = (sem_sc_0, sem_sc_1)

    def task_body(i, carry):
        b = i // (_NCB // _NC)
        cb = 2 * lax.rem(i, _NCB // _NC) + cid

        @pl.when(i == 0)
        def _():
            for k in range(_NCHUNK):
                pltpu.async_copy(
                    zeros, table.at[pl.ds(tile_p0 + k * _CHUNK, _CHUNK), :],
                    sem_misc)
            for k in range(_NCHUNK):
                pltpu.make_async_copy(
                    zeros, table.at[pl.ds(tile_p0, _CHUNK), :],
                    sem_misc).wait()

        def issue_inputs(k, kk):
            pbase = tile_p0 + k * _CHUNK
            row0 = tile_r0 + k * _RC
            ht = row0 // 8
            rr = lax.rem(row0, 8)
            pltpu.async_copy(
                frame.at[b, pl.ds(cb * _CB, _CB), ht, :, pl.ds(rr, _RC), :],
                vraws[kk], sems_in[kk])
            pltpu.async_copy(
                wgt.at[b, :, pl.ds(pbase, _CHUNK)], wbs[kk], sems_in[kk])
            pltpu.async_copy(
                idxr.at[b, :, pl.ds(pbase, _CHUNK)], idxb.at[k % 3],
                sems_in[kk])

        def wait_inputs(kk):
            pltpu.make_async_copy(
                frame.at[b, pl.ds(cb * _CB, _CB), 0, :, pl.ds(0, _RC), :],
                vraws[kk], sems_in[kk]).wait()
            pltpu.make_async_copy(
                wgt.at[b, :, pl.ds(tile_p0, _CHUNK)], wbs[kk],
                sems_in[kk]).wait()
            pltpu.make_async_copy(
                idxr.at[b, :, pl.ds(tile_p0, _CHUNK)], idxb.at[0],
                sems_in[kk]).wait()

        def drain_tap(t):
            pltpu.make_async_copy(
                valss[t % 2], table.at[idxb.at[0, t]],
                sems_sc[t % 2]).wait()

        def compute_tap(kk, t):
            vraw, wb, vals = vraws[kk], wbs[kk], valss[t % 2]

            for r in range(_RC):
                def cgroup(g, cc, r=r):
                    col = g * 16
                    wt = g // 8
                    cc128 = lax.rem(col, 128)
                    rb = r * _W + col
                    rowv = i16 + rb
                    wv = wb[t, pl.ds(rb, 16)]
                    for ch in range(_CB):
                        v = vraw[ch, wt, r, pl.ds(cc128, 16)]
                        plsc.store_scatter(vals, [rowv, cols[ch]], v * wv)
                    return cc

                lax.fori_loop(0, _G16, cgroup, 0)

        def fire_tap(k, t):
            pltpu.async_copy(
                valss[t % 2],
                table.at[idxb.at[k % 3, t]],
                sems_sc[t % 2],
                add=True)

        issue_inputs(0, 0)
        issue_inputs(1, 1)
        plsc.subcore_barrier()

        def chunk_pair(m, c):
            for kk in range(2):
                k = 2 * m + kk
                wait_inputs(kk)
                for t in range(4):
                    if t < 2:
                        @pl.when(k >= 1)
                        def _():
                            drain_tap(t)
                    else:
                        drain_tap(t)
                    compute_tap(kk, t)
                    fire_tap(k, t)

                @pl.when(k + 2 < _NCHUNK)
                def _():
                    issue_inputs(k + 2, kk)
            return c

        lax.fori_loop(0, _NCHUNK // 2, chunk_pair, 0)
        drain_tap(2)
        drain_tap(3)
        plsc.subcore_barrier()

        def issue_tin(k, kk):
            pltpu.async_copy(
                table.at[pl.ds(tile_p0 + k * _CHUNK, _CHUNK), :],
                valss[kk], sems_in[kk])

        def wait_tin(kk):
            pltpu.make_async_copy(
                table.at[pl.ds(tile_p0, _CHUNK), :], valss[kk],
                sems_in[kk]).wait()

        def fire_tout(k, kk):
            row0 = tile_r0 + k * _RC
            ht = row0 // 8
            rr = lax.rem(row0, 8)
            pltpu.async_copy(
                vraws[kk],
                out.at[b, pl.ds(cb * _CB, _CB), ht, :, pl.ds(rr, _RC), :],
                sems_sc[kk])

        def wait_tout(kk):
            pltpu.make_async_copy(
                vraws[kk],
                out.at[b, pl.ds(cb * _CB, _CB), 0, :, pl.ds(0, _RC), :],
                sems_sc[kk]).wait()

        issue_tin(0, 0)

        def drain_pair(m, c):
            for kk in range(2):
                k = 2 * m + kk

                @pl.when(k + 1 < _NCHUNK)
                def _():
                    issue_tin(k + 1, 1 - kk)

                wait_tin(kk)
                pltpu.async_copy(
                    zeros,
                    table.at[pl.ds(tile_p0 + k * _CHUNK, _CHUNK), :],
                    sem_misc)

                @pl.when(k >= 2)
                def _():
                    wait_tout(kk)

                obuf, obuft = valss[kk], vraws[kk]
                for r in range(_RC):
                    def tgroup(g, cc, r=r):
                        col = g * 16
                        wt = g // 8
                        cc128 = lax.rem(col, 128)
                        rows = i16 + (r * _W + col)
                        for ch in range(_CB):
                            v = plsc.load_gather(obuf, [rows, cols[ch]])
                            obuft[ch, wt, r, pl.ds(cc128, 16)] = v
                        return cc

                    lax.fori_loop(0, _G16, tgroup, 0)
                fire_tout(k, kk)
            return c

        lax.fori_loop(0, _NCHUNK // 2, drain_pair, 0)
        wait_tout(0)
        wait_tout(1)
        for k in range(_NCHUNK):
            pltpu.make_async_copy(
                zeros, table.at[pl.ds(tile_p0, _CHUNK), :], sem_misc).wait()
        return carry

    lax.fori_loop(0, _B * _NCB // _NC, task_body, 0)


_splat_sc = functools.partial(
    pl.kernel,
    out_type=jax.ShapeDtypeStruct((_B, _C, _H // 8, _W // 128, 8, 128),
                                  jnp.float32),
    mesh=plsc.VectorSubcoreMesh(core_axis_name="c", subcore_axis_name="s"),
    scratch_types=[
        pltpu.VMEM_SHARED((_HW, _CB), jnp.float32),
        pltpu.VMEM((_CB, _W // 128, _RC, 128), jnp.float32),
        pltpu.VMEM((_CB, _W // 128, _RC, 128), jnp.float32),
        pltpu.VMEM((4, _CHUNK), jnp.float32),
        pltpu.VMEM((4, _CHUNK), jnp.float32),
        pltpu.VMEM((3, 4, _CHUNK), jnp.int32),
        pltpu.VMEM((_CHUNK, _CB), jnp.float32),
        pltpu.VMEM((_CHUNK, _CB), jnp.float32),
        pltpu.SemaphoreType.DMA,
        pltpu.SemaphoreType.DMA,
        pltpu.SemaphoreType.DMA,
        pltpu.SemaphoreType.DMA,
        pltpu.SemaphoreType.DMA,
    ],
    compiler_params=pltpu.CompilerParams(
        needs_layout_passes=False, use_tc_tiling_on_sc=False),
)(_splat_sc_body)


@jax.jit
def kernel(frame, flow):
    idx, wgt = _taps(flow)
    fr6 = frame.reshape(_B, _C, _H // 8, 8, _W // 128,
                        128).transpose(0, 1, 2, 4, 3, 5)
    out6 = _splat_sc(
        fr6,
        idx.reshape(_B, 4, _HW),
        wgt.reshape(_B, 4, _HW),
        jnp.zeros((_CHUNK, _CB), jnp.float32),
    )
    return out6.transpose(0, 1, 2, 4, 3, 5).reshape(_B, _C, _H, _W)
